# Initial kernel scaffold; baseline (speedup 1.0000x reference)
import jax
import jax.numpy as jnp
from jax.experimental import pallas as pl


def kernel(x, edge_index, W1l, b1, W1r, W2l, b2, W2r, W3l, b3, W3r, W4l, b4, W4r):
    raise NotImplementedError("write your pallas kernel here")


# speedup vs baseline: 3.0687x; 3.0687x over previous
import functools

import jax
import jax.numpy as jnp
from jax import lax
from jax.experimental import pallas as pl
from jax.experimental.pallas import tpu as pltpu
from jax.experimental.pallas import tpu_sc as plsc

N = 10000
E = 320000
D_IN = 128
D_HID = 128
D_LAT = 64

NT = N + 8
NCORE = 2
NSUB = 16
NW = NCORE * NSUB
RPT = 632
N_ACC = NSUB * RPT
B = 128
EP = NW * 80 * B
TPB = EP // NW
ITERS = TPB // B

_f32 = jnp.float32


def _make_agg(D):
    mesh = plsc.VectorSubcoreMesh(core_axis_name="c", subcore_axis_name="s")
    out_type = jax.ShapeDtypeStruct((NCORE, N_ACC, D), _f32)
    scratch = [
        pltpu.VMEM((B,), jnp.int32),
        pltpu.VMEM((B,), jnp.int32),
        pltpu.VMEM((B, D), _f32),
        pltpu.VMEM_SHARED((N_ACC, D), _f32),
        pltpu.SemaphoreType.DMA,
    ]

    def body(table, srcp, dstp, zeros2, out, sidx, didx, rows, acc, sem):
        c = lax.axis_index("c")
        s = lax.axis_index("s")
        w = c * NSUB + s
        rbase = s * RPT
        pltpu.sync_copy(zeros2.at[pl.ds(rbase, RPT)], acc.at[pl.ds(rbase, RPT)])
        plsc.subcore_barrier()

        def step(i, carry):
            ebase = w * TPB + i * B
            pltpu.sync_copy(srcp.at[pl.ds(ebase, B)], sidx)
            pltpu.sync_copy(dstp.at[pl.ds(ebase, B)], didx)
            pltpu.async_copy(table.at[sidx], rows, sem).wait()
            pltpu.sync_copy(rows, acc.at[didx], add=True)
            return carry

        lax.fori_loop(0, ITERS, step, 0)
        plsc.subcore_barrier()
        pltpu.sync_copy(acc.at[pl.ds(rbase, RPT)], out.at[c, pl.ds(rbase, RPT)])

    return pl.kernel(body, out_type=out_type, mesh=mesh, scratch_types=scratch,
                     compiler_params=pltpu.CompilerParams(use_tc_tiling_on_sc=False),
                     name=f"sc_agg_d{D}")


D_AUG = 144
_agg144 = _make_agg(D_AUG)
_agg128 = _make_agg(128)
_agg64 = _make_agg(64)


def _k1_body(agg, x, w1l, b1, w1r, w2l, h_ref, m2_ref, inv_ref):
    cnt = agg[0, :N, 128] + agg[1, :N, 128]
    inv = 1.0 / jnp.maximum(cnt, 1.0)
    inv_ref[...] = inv
    a = (agg[0, :N, :128] + agg[1, :N, :128]) * inv[:, None]
    h = a @ w1l[...] + b1[...][None, :] + x[...] @ w1r[...]
    h = jnp.maximum(h, 0.0)
    h_ref[:N, :] = h
    h_ref[N:, :] = jnp.zeros((NT - N, D_HID), _f32)
    m2_ref[:N, :] = h @ w2l[...]
    m2_ref[N:, :] = jnp.zeros((NT - N, D_LAT), _f32)


def _k2_body(s2, inv, h, w2r, b2, z_ref):
    z = ((s2[0, :N, :] + s2[1, :N, :]) * inv[...][:, None]
         + b2[...][None, :] + h[:N, :] @ w2r[...])
    z_ref[:N, :] = z
    z_ref[N:, :] = jnp.zeros((NT - N, D_LAT), _f32)


def _k3_body(agg3, inv, z, w3l, b3, w3r, w4l, h2_ref, m4_ref):
    a = (agg3[0, :N, :] + agg3[1, :N, :]) * inv[...][:, None]
    h2 = a @ w3l[...] + b3[...][None, :] + z[:N, :] @ w3r[...]
    h2 = jnp.maximum(h2, 0.0)
    h2_ref[:N, :] = h2
    h2_ref[N:, :] = jnp.zeros((NT - N, D_HID), _f32)
    m4_ref[:N, :] = h2 @ w4l[...]
    m4_ref[N:, :] = jnp.zeros((NT - N, D_IN), _f32)


def _k4_body(s4, inv, h2, w4r, b4, out_ref):
    out_ref[...] = ((s4[0, :N, :] + s4[1, :N, :]) * inv[...][:, None]
                    + b4[...][None, :] + h2[:N, :] @ w4r[...])


_k1 = pl.pallas_call(
    _k1_body,
    out_shape=[jax.ShapeDtypeStruct((NT, D_HID), _f32),
               jax.ShapeDtypeStruct((NT, D_LAT), _f32),
               jax.ShapeDtypeStruct((N,), _f32)])
_k2 = pl.pallas_call(
    _k2_body,
    out_shape=jax.ShapeDtypeStruct((NT, D_LAT), _f32))
_k3 = pl.pallas_call(
    _k3_body,
    out_shape=[jax.ShapeDtypeStruct((NT, D_HID), _f32),
               jax.ShapeDtypeStruct((NT, D_IN), _f32)])
_k4 = pl.pallas_call(
    _k4_body,
    out_shape=jax.ShapeDtypeStruct((N, D_IN), _f32))


def kernel(x, edge_index, W1l, b1, W1r, W2l, b2, W2r, W3l, b3, W3r, W4l, b4, W4r):
    src = edge_index[0]
    dst = edge_index[1]
    pad = jnp.full((EP - E,), N, jnp.int32)
    srcp = jnp.concatenate([src, pad])
    dstp = jnp.concatenate([dst, pad])
    xt = jnp.concatenate([x, jnp.zeros((NT - N, D_IN), _f32)])
    zeros2_144 = jnp.zeros((N_ACC, D_AUG), _f32)
    zeros2_128 = jnp.zeros((N_ACC, 128), _f32)
    zeros2_64 = jnp.zeros((N_ACC, 64), _f32)
    xa = jnp.concatenate(
        [xt, jnp.concatenate([jnp.ones((N, 1), _f32), jnp.zeros((NT - N, 1), _f32)]),
         jnp.zeros((NT, D_AUG - 129), _f32)], axis=1)

    agg1 = _agg144(xa, srcp, dstp, zeros2_144)
    h, m2, inv = _k1(agg1, x, W1l, b1, W1r, W2l)
    s2 = _agg64(m2, srcp, dstp, zeros2_64)
    z = _k2(s2, inv, h, W2r, b2)
    agg3 = _agg64(z, srcp, dstp, zeros2_64)
    h2, m4 = _k3(agg3, inv, z, W3l, b3, W3r, W4l)
    s4 = _agg128(m4, srcp, dstp, zeros2_128)
    x_hat = _k4(s4, inv, h2, W4r, b4)
    return x_hat


# speedup vs baseline: 3.6134x; 1.1775x over previous
import functools

import jax
import jax.numpy as jnp
from jax import lax
from jax.experimental import pallas as pl
from jax.experimental.pallas import tpu as pltpu
from jax.experimental.pallas import tpu_sc as plsc

N = 10000
E = 320000
D_IN = 128
D_HID = 128
D_LAT = 64

NT = N + 8
NCORE = 2
NSUB = 16
NW = NCORE * NSUB
RPT = 632
N_ACC = NSUB * RPT
B = 128
EP = NW * 80 * B
TPB = EP // NW
ITERS = TPB // B

_f32 = jnp.float32


def _make_agg(D, Bx, NBUFx):
    iters = TPB // Bx
    mesh = plsc.VectorSubcoreMesh(core_axis_name="c", subcore_axis_name="s")
    out_type = jax.ShapeDtypeStruct((NCORE, N_ACC, D), _f32)
    scratch = (
        [pltpu.VMEM((Bx,), jnp.int32) for _ in range(NBUFx)]
        + [pltpu.VMEM((Bx,), jnp.int32) for _ in range(NBUFx)]
        + [pltpu.VMEM((Bx, D), _f32) for _ in range(NBUFx)]
        + [pltpu.VMEM_SHARED((N_ACC, D), _f32)]
        + [pltpu.SemaphoreType.DMA for _ in range(3 * NBUFx)]
    )

    def body(table, src1, dst1, zeros2, out, *rest):
        sidx = rest[:NBUFx]
        didx = rest[NBUFx:2 * NBUFx]
        rows = rest[2 * NBUFx:3 * NBUFx]
        acc = rest[3 * NBUFx]
        sems = rest[3 * NBUFx + 1:]
        isem = sems[:NBUFx]
        jsem = sems[NBUFx:2 * NBUFx]
        gsem = sems[2 * NBUFx:3 * NBUFx]
        c = lax.axis_index("c")
        s = lax.axis_index("s")
        w = c * NSUB + s
        rbase = s * RPT
        pltpu.sync_copy(zeros2.at[pl.ds(rbase, RPT)], acc.at[pl.ds(rbase, RPT)])
        plsc.subcore_barrier()

        def step(j, carry):
            e0 = w * TPB + j * NBUFx * Bx
            ia = [pltpu.async_copy(src1.at[pl.ds(e0 + b * Bx, Bx)], sidx[b], isem[b])
                  for b in range(NBUFx)]
            ib = [pltpu.async_copy(dst1.at[pl.ds(e0 + b * Bx, Bx)], didx[b], jsem[b])
                  for b in range(NBUFx)]
            gd = []
            for b in range(NBUFx):
                ia[b].wait()
                gd.append(pltpu.async_copy(table.at[sidx[b]], rows[b], gsem[b]))
            for b in range(NBUFx):
                gd[b].wait()
                ib[b].wait()
                pltpu.sync_copy(rows[b], acc.at[didx[b]], add=True)
            return carry

        lax.fori_loop(0, iters // NBUFx, step, 0)
        plsc.subcore_barrier()
        pltpu.sync_copy(acc.at[pl.ds(rbase, RPT)], out.at[c, pl.ds(rbase, RPT)])

    return pl.kernel(body, out_type=out_type, mesh=mesh, scratch_types=scratch,
                     compiler_params=pltpu.CompilerParams(use_tc_tiling_on_sc=False),
                     name=f"sc_agg_d{D}")


D_AUG = 144
_agg144 = _make_agg(D_AUG, 80, 2)
_agg128 = _make_agg(128, 80, 3)
_agg64 = _make_agg(64, 128, 4)


def _k1_body(agg, x, w1l, b1, w1r, w2l, h_ref, m2_ref, inv_ref):
    cnt = agg[0, :N, 128] + agg[1, :N, 128]
    inv = 1.0 / jnp.maximum(cnt, 1.0)
    inv_ref[...] = inv
    a = (agg[0, :N, :128] + agg[1, :N, :128]) * inv[:, None]
    h = a @ w1l[...] + b1[...][None, :] + x[...] @ w1r[...]
    h = jnp.maximum(h, 0.0)
    h_ref[:N, :] = h
    h_ref[N:, :] = jnp.zeros((NT - N, D_HID), _f32)
    m2_ref[:N, :] = h @ w2l[...]
    m2_ref[N:, :] = jnp.zeros((NT - N, D_LAT), _f32)


def _k2_body(s2, inv, h, w2r, b2, z_ref):
    z = ((s2[0, :N, :] + s2[1, :N, :]) * inv[...][:, None]
         + b2[...][None, :] + h[:N, :] @ w2r[...])
    z_ref[:N, :] = z
    z_ref[N:, :] = jnp.zeros((NT - N, D_LAT), _f32)


def _k3_body(agg3, inv, z, w3l, b3, w3r, w4l, h2_ref, m4_ref):
    a = (agg3[0, :N, :] + agg3[1, :N, :]) * inv[...][:, None]
    h2 = a @ w3l[...] + b3[...][None, :] + z[:N, :] @ w3r[...]
    h2 = jnp.maximum(h2, 0.0)
    h2_ref[:N, :] = h2
    h2_ref[N:, :] = jnp.zeros((NT - N, D_HID), _f32)
    m4_ref[:N, :] = h2 @ w4l[...]
    m4_ref[N:, :] = jnp.zeros((NT - N, D_IN), _f32)


def _k4_body(s4, inv, h2, w4r, b4, out_ref):
    out_ref[...] = ((s4[0, :N, :] + s4[1, :N, :]) * inv[...][:, None]
                    + b4[...][None, :] + h2[:N, :] @ w4r[...])


_k1 = pl.pallas_call(
    _k1_body,
    out_shape=[jax.ShapeDtypeStruct((NT, D_HID), _f32),
               jax.ShapeDtypeStruct((NT, D_LAT), _f32),
               jax.ShapeDtypeStruct((N,), _f32)])
_k2 = pl.pallas_call(
    _k2_body,
    out_shape=jax.ShapeDtypeStruct((NT, D_LAT), _f32))
_k3 = pl.pallas_call(
    _k3_body,
    out_shape=[jax.ShapeDtypeStruct((NT, D_HID), _f32),
               jax.ShapeDtypeStruct((NT, D_IN), _f32)])
_k4 = pl.pallas_call(
    _k4_body,
    out_shape=jax.ShapeDtypeStruct((N, D_IN), _f32))


def kernel(x, edge_index, W1l, b1, W1r, W2l, b2, W2r, W3l, b3, W3r, W4l, b4, W4r):
    src = edge_index[0]
    dst = edge_index[1]
    pad = jnp.full((EP - E,), N, jnp.int32)
    srcp = jnp.concatenate([src, pad])
    dstp = jnp.concatenate([dst, pad])
    xt = jnp.concatenate([x, jnp.zeros((NT - N, D_IN), _f32)])
    zeros2_144 = jnp.zeros((N_ACC, D_AUG), _f32)
    zeros2_128 = jnp.zeros((N_ACC, 128), _f32)
    zeros2_64 = jnp.zeros((N_ACC, 64), _f32)
    xa = jnp.concatenate(
        [xt, jnp.concatenate([jnp.ones((N, 1), _f32), jnp.zeros((NT - N, 1), _f32)]),
         jnp.zeros((NT, D_AUG - 129), _f32)], axis=1)

    agg1 = _agg144(xa, srcp, dstp, zeros2_144)
    h, m2, inv = _k1(agg1, x, W1l, b1, W1r, W2l)
    s2 = _agg64(m2, srcp, dstp, zeros2_64)
    z = _k2(s2, inv, h, W2r, b2)
    agg3 = _agg64(z, srcp, dstp, zeros2_64)
    h2, m4 = _k3(agg3, inv, z, W3l, b3, W3r, W4l)
    s4 = _agg128(m4, srcp, dstp, zeros2_128)
    x_hat = _k4(s4, inv, h2, W4r, b4)
    return x_hat
